# baseline (speedup 1.0000x reference)
import functools

import jax
import jax.numpy as jnp
from jax import lax
from jax.experimental import pallas as pl
from jax.experimental.pallas import tpu as pltpu
from jax.experimental.pallas import tpu_sc as plsc

_B, _S, _D = 4, 8192, 1024
_E, _R = 64, 16
_RH = 256
_ALPHA = 32
_TEMP = 1.0
_BALANCE_WEIGHT = 0.1
_SCALE = _ALPHA / _R

_SB1 = 1024
_SB3 = 1024


def _pool_router_body(h_ref, m_ref, w1_ref, b1_ref, w2_ref, b2_ref,
                      rw_ref, loss_ref, acc_ref, ms_ref):
    i = pl.program_id(0)
    nsb = pl.num_programs(0)

    @pl.when(i == 0)
    def _init():
        acc_ref[...] = jnp.zeros_like(acc_ref)
        ms_ref[...] = jnp.zeros_like(ms_ref)

    m = m_ref[...]
    acc_ref[...] += jnp.sum(h_ref[...] * m[..., None], axis=1)
    ms_ref[...] += jnp.sum(m, axis=1, keepdims=True)

    @pl.when(i == nsb - 1)
    def _finish():
        denom = jnp.clip(ms_ref[:, :1], 1e-6, None)
        pooled = acc_ref[...] / denom
        h1 = jnp.maximum(
            jnp.dot(pooled, w1_ref[...],
                    preferred_element_type=jnp.float32) + b1_ref[...], 0.0)
        logits = (jnp.dot(h1, w2_ref[...],
                          preferred_element_type=jnp.float32)
                  + b2_ref[...]) * (1.0 / _TEMP)
        z = logits - jnp.max(logits, axis=1, keepdims=True)
        ez = jnp.exp(z)
        rw = ez / jnp.sum(ez, axis=1, keepdims=True)
        rw_ref[...] = rw
        imp = jnp.mean(rw, axis=0, keepdims=True)
        loss = _BALANCE_WEIGHT * jnp.mean((imp * _E - 1.0) ** 2)
        loss_ref[...] = jnp.broadcast_to(loss, (1, 1))


def _pool_router(hidden_states, attention_mask, W1, b1, W2, b2):
    nsb = _S // _SB1
    return pl.pallas_call(
        _pool_router_body,
        grid=(nsb,),
        in_specs=[
            pl.BlockSpec((_B, _SB1, _D), lambda i: (0, i, 0)),
            pl.BlockSpec((_B, _SB1), lambda i: (0, i)),
            pl.BlockSpec((_D, _RH), lambda i: (0, 0)),
            pl.BlockSpec((1, _RH), lambda i: (0, 0)),
            pl.BlockSpec((_RH, _E), lambda i: (0, 0)),
            pl.BlockSpec((1, _E), lambda i: (0, 0)),
        ],
        out_specs=[
            pl.BlockSpec((_B, _E), lambda i: (0, 0)),
            pl.BlockSpec((1, 1), lambda i: (0, 0)),
        ],
        out_shape=[
            jax.ShapeDtypeStruct((_B, _E), jnp.float32),
            jax.ShapeDtypeStruct((1, 1), jnp.float32),
        ],
        scratch_shapes=[
            pltpu.VMEM((_B, _D), jnp.float32),
            pltpu.VMEM((_B, 128), jnp.float32),
        ],
        compiler_params=pltpu.CompilerParams(
            dimension_semantics=("arbitrary",)),
    )(hidden_states, attention_mask, W1, b1.reshape(1, _RH),
      W2, b2.reshape(1, _E))


_LANES = 16
_CHUNK = _R * _D // 4


def _sc_topk_rows(rw_v):
    lane = lax.iota(jnp.int32, _LANES)
    ivec = jnp.zeros((_LANES,), jnp.int32)
    gvec = jnp.zeros((_LANES,), jnp.int32)
    for b in range(_B):
        val = rw_v[b, pl.ds(0, _LANES)]
        idxv = lane
        for c in range(1, _E // _LANES):
            v = rw_v[b, pl.ds(c * _LANES, _LANES)]
            iv = lane + c * _LANES
            take = v > val
            val = jnp.where(take, v, val)
            idxv = jnp.where(take, iv, idxv)
        for sh in (8, 4, 2, 1):
            perm = lane ^ sh
            ov = val.at[perm].get(mode="promise_in_bounds")
            oi = idxv.at[perm].get(mode="promise_in_bounds")
            take = (ov > val) | ((ov == val) & (oi < idxv))
            val = jnp.where(take, ov, val)
            idxv = jnp.where(take, oi, idxv)
        ivec = jnp.where(lane == b, idxv, ivec)
        gvec = jnp.where((lane >> 2) == b, idxv, gvec)
    gidx = gvec * 4 + (lane & 3)
    return ivec, gidx


def _sc_gather_kernel(rw_hbm, a_hbm, b_hbm, asel_hbm, bsel_hbm, idx_hbm,
                      rw_v, idx_v, rows_v, sem):
    w = lax.axis_index("s") * 2 + lax.axis_index("c")

    @pl.when(w == 0)
    def _gather_a():
        pltpu.sync_copy(rw_hbm, rw_v)
        ivec, gidx = _sc_topk_rows(rw_v)
        idx_v[...] = gidx
        pltpu.async_copy(a_hbm.at[idx_v], rows_v, sem).wait()
        pltpu.sync_copy(rows_v, asel_hbm)
        idx_v[...] = ivec
        pltpu.sync_copy(idx_v, idx_hbm)

    @pl.when(w == 1)
    def _gather_b():
        pltpu.sync_copy(rw_hbm, rw_v)
        _, gidx = _sc_topk_rows(rw_v)
        idx_v[...] = gidx
        pltpu.async_copy(b_hbm.at[idx_v], rows_v, sem).wait()
        pltpu.sync_copy(rows_v, bsel_hbm)


def _sc_gather(rw, a4, b4):
    mesh = plsc.VectorSubcoreMesh(core_axis_name="c", subcore_axis_name="s")
    fn = functools.partial(
        pl.kernel,
        mesh=mesh,
        out_type=[
            jax.ShapeDtypeStruct((_LANES, _CHUNK), jnp.float32),
            jax.ShapeDtypeStruct((_LANES, _CHUNK), jnp.float32),
            jax.ShapeDtypeStruct((_LANES,), jnp.int32),
        ],
        scratch_types=[
            pltpu.VMEM((_B, _E), jnp.float32),
            pltpu.VMEM((_LANES,), jnp.int32),
            pltpu.VMEM((_LANES, _CHUNK), jnp.float32),
            pltpu.SemaphoreType.DMA,
        ],
    )(_sc_gather_kernel)
    return fn(rw, a4, b4)


def _apply_body(h_ref, a_ref, b_ref, o_ref):
    h = h_ref[0]
    low = lax.dot_general(h, a_ref[0], (((1,), (1,)), ((), ())),
                          preferred_element_type=jnp.float32)
    delta = lax.dot_general(low, b_ref[0], (((1,), (1,)), ((), ())),
                            preferred_element_type=jnp.float32)
    o_ref[0] = h + _SCALE * delta


def _apply(hidden_states, a_sel, b_sel):
    nsb = _S // _SB3
    return pl.pallas_call(
        _apply_body,
        grid=(_B, nsb),
        in_specs=[
            pl.BlockSpec((1, _SB3, _D), lambda b, s: (b, s, 0)),
            pl.BlockSpec((1, _R, _D), lambda b, s: (b, 0, 0)),
            pl.BlockSpec((1, _D, _R), lambda b, s: (b, 0, 0)),
        ],
        out_specs=pl.BlockSpec((1, _SB3, _D), lambda b, s: (b, s, 0)),
        out_shape=jax.ShapeDtypeStruct((_B, _S, _D), jnp.float32),
        compiler_params=pltpu.CompilerParams(
            dimension_semantics=("arbitrary", "arbitrary")),
    )(hidden_states, a_sel, b_sel)


def kernel(input_ids, attention_mask, hidden_states, W1, b1, W2, b2,
           lora_A, lora_B):
    del input_ids
    rw, loss11 = _pool_router(hidden_states, attention_mask, W1, b1, W2, b2)
    a4 = lora_A.reshape(_E * 4, _CHUNK)
    b4 = lora_B.reshape(_E * 4, _CHUNK)
    asel, bsel, idx16 = _sc_gather(rw, a4, b4)
    a_sel = asel.reshape(_B, _R, _D)
    b_sel = bsel.reshape(_B, _D, _R)
    out = _apply(hidden_states, a_sel, b_sel)
    top_k_indices = idx16[:_B].reshape(_B, 1)
    balance_loss = loss11[0, 0]
    return out, rw, top_k_indices, balance_loss


# speedup vs baseline: 1.0310x; 1.0310x over previous
import functools

import jax
import jax.numpy as jnp
from jax import lax
from jax.experimental import pallas as pl
from jax.experimental.pallas import tpu as pltpu
from jax.experimental.pallas import tpu_sc as plsc

_B, _S, _D = 4, 8192, 1024
_E, _R = 64, 16
_RH = 256
_ALPHA = 32
_TEMP = 1.0
_BALANCE_WEIGHT = 0.1
_SCALE = _ALPHA / _R

_SB1 = 1024
_SB3 = 2048


def _pool_router_body(h_ref, m_ref, w1_ref, b1_ref, w2_ref, b2_ref,
                      rw_ref, loss_ref, acc_ref, ms_ref):
    i = pl.program_id(0)
    nsb = pl.num_programs(0)

    @pl.when(i == 0)
    def _init():
        acc_ref[...] = jnp.zeros_like(acc_ref)
        ms_ref[...] = jnp.zeros_like(ms_ref)

    m = m_ref[...]
    acc_ref[...] += lax.dot_general(
        m, h_ref[...], (((1,), (1,)), ((0,), (0,))),
        preferred_element_type=jnp.float32)
    ms_ref[...] += jnp.sum(m, axis=1, keepdims=True)

    @pl.when(i == nsb - 1)
    def _finish():
        denom = jnp.clip(ms_ref[:, :1], 1e-6, None)
        pooled = acc_ref[...] / denom
        h1 = jnp.maximum(
            jnp.dot(pooled, w1_ref[...],
                    preferred_element_type=jnp.float32) + b1_ref[...], 0.0)
        logits = (jnp.dot(h1, w2_ref[...],
                          preferred_element_type=jnp.float32)
                  + b2_ref[...]) * (1.0 / _TEMP)
        z = logits - jnp.max(logits, axis=1, keepdims=True)
        ez = jnp.exp(z)
        rw = ez / jnp.sum(ez, axis=1, keepdims=True)
        rw_ref[...] = rw
        imp = jnp.mean(rw, axis=0, keepdims=True)
        loss = _BALANCE_WEIGHT * jnp.mean((imp * _E - 1.0) ** 2)
        loss_ref[...] = jnp.broadcast_to(loss, (1, 1))


def _pool_router(hidden_states, attention_mask, W1, b1, W2, b2):
    nsb = _S // _SB1
    return pl.pallas_call(
        _pool_router_body,
        grid=(nsb,),
        in_specs=[
            pl.BlockSpec((_B, _SB1, _D), lambda i: (0, i, 0)),
            pl.BlockSpec((_B, _SB1), lambda i: (0, i)),
            pl.BlockSpec((_D, _RH), lambda i: (0, 0)),
            pl.BlockSpec((1, _RH), lambda i: (0, 0)),
            pl.BlockSpec((_RH, _E), lambda i: (0, 0)),
            pl.BlockSpec((1, _E), lambda i: (0, 0)),
        ],
        out_specs=[
            pl.BlockSpec((_B, _E), lambda i: (0, 0)),
            pl.BlockSpec((1, 1), lambda i: (0, 0)),
        ],
        out_shape=[
            jax.ShapeDtypeStruct((_B, _E), jnp.float32),
            jax.ShapeDtypeStruct((1, 1), jnp.float32),
        ],
        scratch_shapes=[
            pltpu.VMEM((_B, _D), jnp.float32),
            pltpu.VMEM((_B, 128), jnp.float32),
        ],
        compiler_params=pltpu.CompilerParams(
            dimension_semantics=("arbitrary",)),
    )(hidden_states, attention_mask, W1, b1.reshape(1, _RH),
      W2, b2.reshape(1, _E))


_LANES = 16
_CHUNK = _R * _D // 4


def _sc_topk_rows(rw_v):
    lane = lax.iota(jnp.int32, _LANES)
    ivec = jnp.zeros((_LANES,), jnp.int32)
    gvec = jnp.zeros((_LANES,), jnp.int32)
    for b in range(_B):
        val = rw_v[b, pl.ds(0, _LANES)]
        idxv = lane
        for c in range(1, _E // _LANES):
            v = rw_v[b, pl.ds(c * _LANES, _LANES)]
            iv = lane + c * _LANES
            take = v > val
            val = jnp.where(take, v, val)
            idxv = jnp.where(take, iv, idxv)
        for sh in (8, 4, 2, 1):
            perm = lane ^ sh
            ov = val.at[perm].get(mode="promise_in_bounds")
            oi = idxv.at[perm].get(mode="promise_in_bounds")
            take = (ov > val) | ((ov == val) & (oi < idxv))
            val = jnp.where(take, ov, val)
            idxv = jnp.where(take, oi, idxv)
        ivec = jnp.where(lane == b, idxv, ivec)
        gvec = jnp.where((lane >> 2) == b, idxv, gvec)
    gidx = gvec * 4 + (lane & 3)
    return ivec, gidx


def _sc_gather_kernel(rw_hbm, a_hbm, b_hbm, asel_hbm, bsel_hbm, idx_hbm,
                      rw_v, idx_v, rows_v, sem):
    w = lax.axis_index("s") * 2 + lax.axis_index("c")

    @pl.when(w == 0)
    def _gather_a():
        pltpu.sync_copy(rw_hbm, rw_v)
        ivec, gidx = _sc_topk_rows(rw_v)
        idx_v[...] = gidx
        pltpu.async_copy(a_hbm.at[idx_v], rows_v, sem).wait()
        pltpu.sync_copy(rows_v, asel_hbm)
        idx_v[...] = ivec
        pltpu.sync_copy(idx_v, idx_hbm)

    @pl.when(w == 1)
    def _gather_b():
        pltpu.sync_copy(rw_hbm, rw_v)
        _, gidx = _sc_topk_rows(rw_v)
        idx_v[...] = gidx
        pltpu.async_copy(b_hbm.at[idx_v], rows_v, sem).wait()
        pltpu.sync_copy(rows_v, bsel_hbm)


def _sc_gather(rw, a4, b4):
    mesh = plsc.VectorSubcoreMesh(core_axis_name="c", subcore_axis_name="s")
    fn = functools.partial(
        pl.kernel,
        mesh=mesh,
        out_type=[
            jax.ShapeDtypeStruct((_LANES, _CHUNK), jnp.float32),
            jax.ShapeDtypeStruct((_LANES, _CHUNK), jnp.float32),
            jax.ShapeDtypeStruct((_LANES,), jnp.int32),
        ],
        scratch_types=[
            pltpu.VMEM((_B, _E), jnp.float32),
            pltpu.VMEM((_LANES,), jnp.int32),
            pltpu.VMEM((_LANES, _CHUNK), jnp.float32),
            pltpu.SemaphoreType.DMA,
        ],
    )(_sc_gather_kernel)
    return fn(rw, a4, b4)


def _apply_body(h_ref, a_ref, b_ref, o_ref):
    h = h_ref[0]
    low = lax.dot_general(h, a_ref[0], (((1,), (1,)), ((), ())),
                          preferred_element_type=jnp.float32)
    delta = lax.dot_general(low, b_ref[0], (((1,), (1,)), ((), ())),
                            preferred_element_type=jnp.float32)
    o_ref[0] = h + _SCALE * delta


def _apply(hidden_states, a_sel, b_sel):
    nsb = _S // _SB3
    return pl.pallas_call(
        _apply_body,
        grid=(_B, nsb),
        in_specs=[
            pl.BlockSpec((1, _SB3, _D), lambda b, s: (b, s, 0)),
            pl.BlockSpec((1, _R, _D), lambda b, s: (b, 0, 0)),
            pl.BlockSpec((1, _D, _R), lambda b, s: (b, 0, 0)),
        ],
        out_specs=pl.BlockSpec((1, _SB3, _D), lambda b, s: (b, s, 0)),
        out_shape=jax.ShapeDtypeStruct((_B, _S, _D), jnp.float32),
        compiler_params=pltpu.CompilerParams(
            dimension_semantics=("arbitrary", "arbitrary")),
    )(hidden_states, a_sel, b_sel)


def kernel(input_ids, attention_mask, hidden_states, W1, b1, W2, b2,
           lora_A, lora_B):
    del input_ids
    rw, loss11 = _pool_router(hidden_states, attention_mask, W1, b1, W2, b2)
    a4 = lora_A.reshape(_E * 4, _CHUNK)
    b4 = lora_B.reshape(_E * 4, _CHUNK)
    asel, bsel, idx16 = _sc_gather(rw, a4, b4)
    a_sel = asel.reshape(_B, _R, _D)
    b_sel = bsel.reshape(_B, _D, _R)
    out = _apply(hidden_states, a_sel, b_sel)
    top_k_indices = idx16[:_B].reshape(_B, 1)
    balance_loss = loss11[0, 0]
    return out, rw, top_k_indices, balance_loss


# speedup vs baseline: 1.3098x; 1.2704x over previous
import functools

import jax
import jax.numpy as jnp
from jax import lax
from jax.experimental import pallas as pl
from jax.experimental.pallas import tpu as pltpu
from jax.experimental.pallas import tpu_sc as plsc

_B, _S, _D = 4, 8192, 1024
_E, _R = 64, 16
_RH = 256
_ALPHA = 32
_TEMP = 1.0
_BALANCE_WEIGHT = 0.1
_SCALE = _ALPHA / _R

_SB1 = 1024
_SB3 = 2048


def _pool_router_body(h_ref, m_ref, w1_ref, b1_ref, w2_ref, b2_ref,
                      rw_ref, idx_ref, loss_ref, acc_ref, ms_ref):
    i = pl.program_id(0)
    nsb = pl.num_programs(0)

    @pl.when(i == 0)
    def _init():
        acc_ref[...] = jnp.zeros_like(acc_ref)
        ms_ref[...] = jnp.zeros_like(ms_ref)

    m = m_ref[...]
    acc_ref[...] += lax.dot_general(
        m, h_ref[...], (((1,), (1,)), ((0,), (0,))),
        preferred_element_type=jnp.float32)
    ms_ref[...] += jnp.sum(m, axis=1, keepdims=True)

    @pl.when(i == nsb - 1)
    def _finish():
        denom = jnp.clip(ms_ref[:, :1], 1e-6, None)
        pooled = acc_ref[...] / denom
        h1 = jnp.maximum(
            jnp.dot(pooled, w1_ref[...],
                    preferred_element_type=jnp.float32) + b1_ref[...], 0.0)
        logits = (jnp.dot(h1, w2_ref[...],
                          preferred_element_type=jnp.float32)
                  + b2_ref[...]) * (1.0 / _TEMP)
        z = logits - jnp.max(logits, axis=1, keepdims=True)
        ez = jnp.exp(z)
        rw = ez / jnp.sum(ez, axis=1, keepdims=True)
        rw_ref[...] = rw
        col = lax.broadcasted_iota(jnp.int32, (_B, _E), 1)
        mx = jnp.max(rw, axis=1, keepdims=True)
        idx_ref[...] = jnp.min(jnp.where(rw == mx, col, _E), axis=1,
                               keepdims=True)
        imp = jnp.mean(rw, axis=0, keepdims=True)
        loss = _BALANCE_WEIGHT * jnp.mean((imp * _E - 1.0) ** 2)
        loss_ref[...] = jnp.broadcast_to(loss, (1, 1))


def _pool_router(hidden_states, attention_mask, W1, b1, W2, b2):
    nsb = _S // _SB1
    return pl.pallas_call(
        _pool_router_body,
        grid=(nsb,),
        in_specs=[
            pl.BlockSpec((_B, _SB1, _D), lambda i: (0, i, 0)),
            pl.BlockSpec((_B, _SB1), lambda i: (0, i)),
            pl.BlockSpec((_D, _RH), lambda i: (0, 0)),
            pl.BlockSpec((1, _RH), lambda i: (0, 0)),
            pl.BlockSpec((_RH, _E), lambda i: (0, 0)),
            pl.BlockSpec((1, _E), lambda i: (0, 0)),
        ],
        out_specs=[
            pl.BlockSpec((_B, _E), lambda i: (0, 0)),
            pl.BlockSpec((_B, 1), lambda i: (0, 0)),
            pl.BlockSpec((1, 1), lambda i: (0, 0)),
        ],
        out_shape=[
            jax.ShapeDtypeStruct((_B, _E), jnp.float32),
            jax.ShapeDtypeStruct((_B, 1), jnp.int32),
            jax.ShapeDtypeStruct((1, 1), jnp.float32),
        ],
        scratch_shapes=[
            pltpu.VMEM((_B, _D), jnp.float32),
            pltpu.VMEM((_B, 128), jnp.float32),
        ],
        compiler_params=pltpu.CompilerParams(
            dimension_semantics=("arbitrary",)),
    )(hidden_states, attention_mask, W1, b1.reshape(1, _RH),
      W2, b2.reshape(1, _E))


_LANES = 16


def _sc_top1(rw_v):
    lane = lax.iota(jnp.int32, _LANES)
    ivec = jnp.zeros((_LANES,), jnp.int32)
    for b in range(_B):
        val = rw_v[b, pl.ds(0, _LANES)]
        idxv = lane
        for c in range(1, _E // _LANES):
            v = rw_v[b, pl.ds(c * _LANES, _LANES)]
            iv = lane + c * _LANES
            take = v > val
            val = jnp.where(take, v, val)
            idxv = jnp.where(take, iv, idxv)
        for sh in (8, 4, 2, 1):
            perm = lane ^ sh
            ov = val.at[perm].get(mode="promise_in_bounds")
            oi = idxv.at[perm].get(mode="promise_in_bounds")
            take = (ov > val) | ((ov == val) & (oi < idxv))
            val = jnp.where(take, ov, val)
            idxv = jnp.where(take, oi, idxv)
        ivec = jnp.where(lane == b, idxv, ivec)
    return ivec


def _sc_topk_kernel(rw_hbm, idx_hbm, rw_v, idx_v):
    w = lax.axis_index("s") * 2 + lax.axis_index("c")

    @pl.when(w == 0)
    def _select():
        pltpu.sync_copy(rw_hbm, rw_v)
        idx_v[...] = _sc_top1(rw_v)
        pltpu.sync_copy(idx_v, idx_hbm)


def _sc_topk(rw):
    mesh = plsc.VectorSubcoreMesh(core_axis_name="c", subcore_axis_name="s")
    fn = functools.partial(
        pl.kernel,
        mesh=mesh,
        out_type=jax.ShapeDtypeStruct((_LANES,), jnp.int32),
        scratch_types=[
            pltpu.VMEM((_B, _E), jnp.float32),
            pltpu.VMEM((_LANES,), jnp.int32),
        ],
    )(_sc_topk_kernel)
    return fn(rw)


def _apply_body(sidx_ref, h_ref, a_ref, b_ref, o_ref):
    del sidx_ref
    h = h_ref[0]
    low = lax.dot_general(h, a_ref[0], (((1,), (1,)), ((), ())),
                          preferred_element_type=jnp.float32)
    delta = lax.dot_general(low, b_ref[0], (((1,), (1,)), ((), ())),
                            preferred_element_type=jnp.float32)
    o_ref[0] = h + _SCALE * delta


def _apply(hidden_states, lora_A, lora_B, sidx):
    nsb = _S // _SB3
    grid_spec = pltpu.PrefetchScalarGridSpec(
        num_scalar_prefetch=1,
        grid=(_B, nsb),
        in_specs=[
            pl.BlockSpec((1, _SB3, _D), lambda b, s, sidx: (b, s, 0)),
            pl.BlockSpec((1, _R, _D), lambda b, s, sidx: (sidx[b], 0, 0)),
            pl.BlockSpec((1, _D, _R), lambda b, s, sidx: (sidx[b], 0, 0)),
        ],
        out_specs=pl.BlockSpec((1, _SB3, _D), lambda b, s, sidx: (b, s, 0)),
    )
    return pl.pallas_call(
        _apply_body,
        grid_spec=grid_spec,
        out_shape=jax.ShapeDtypeStruct((_B, _S, _D), jnp.float32),
        compiler_params=pltpu.CompilerParams(
            dimension_semantics=("arbitrary", "arbitrary")),
    )(sidx, hidden_states, lora_A, lora_B)


def kernel(input_ids, attention_mask, hidden_states, W1, b1, W2, b2,
           lora_A, lora_B):
    del input_ids
    rw, idx41, loss11 = _pool_router(hidden_states, attention_mask,
                                     W1, b1, W2, b2)
    idx16 = _sc_topk(rw)
    out = _apply(hidden_states, lora_A, lora_B, idx41.reshape(_B))
    top_k_indices = idx16[:_B].reshape(_B, 1)
    balance_loss = loss11[0, 0]
    return out, rw, top_k_indices, balance_loss


# speedup vs baseline: 1.4365x; 1.0967x over previous
import functools

import jax
import jax.numpy as jnp
from jax import lax
from jax.experimental import pallas as pl
from jax.experimental.pallas import tpu as pltpu
from jax.experimental.pallas import tpu_sc as plsc

_B, _S, _D = 4, 8192, 1024
_E, _R = 64, 16
_RH = 256
_ALPHA = 32
_TEMP = 1.0
_BALANCE_WEIGHT = 0.1
_SCALE = _ALPHA / _R

_SB1 = 1024
_SB3 = 2048


def _pool_router_body(h_ref, m_ref, w1_ref, b1_ref, w2_ref, b2_ref,
                      rw_ref, idx_ref, loss_ref, acc_ref, ms_ref):
    i = pl.program_id(0)
    nsb = pl.num_programs(0)

    @pl.when(i == 0)
    def _init():
        acc_ref[...] = jnp.zeros_like(acc_ref)
        ms_ref[...] = jnp.zeros_like(ms_ref)

    m = m_ref[...]
    acc_ref[...] += lax.dot_general(
        m, h_ref[...], (((1,), (1,)), ((0,), (0,))),
        preferred_element_type=jnp.float32)
    ms_ref[...] += jnp.sum(m, axis=1, keepdims=True)

    @pl.when(i == nsb - 1)
    def _finish():
        denom = jnp.clip(ms_ref[:, :1], 1e-6, None)
        pooled = acc_ref[...] / denom
        h1 = jnp.maximum(
            jnp.dot(pooled, w1_ref[...],
                    preferred_element_type=jnp.float32) + b1_ref[...], 0.0)
        logits = (jnp.dot(h1, w2_ref[...],
                          preferred_element_type=jnp.float32)
                  + b2_ref[...]) * (1.0 / _TEMP)
        z = logits - jnp.max(logits, axis=1, keepdims=True)
        ez = jnp.exp(z)
        rw = ez / jnp.sum(ez, axis=1, keepdims=True)
        rw_ref[...] = rw
        col = lax.broadcasted_iota(jnp.int32, (_B, _E), 1)
        mx = jnp.max(rw, axis=1, keepdims=True)
        idx_ref[...] = jnp.min(jnp.where(rw == mx, col, _E), axis=1,
                               keepdims=True)
        imp = jnp.mean(rw, axis=0, keepdims=True)
        loss = _BALANCE_WEIGHT * jnp.mean((imp * _E - 1.0) ** 2)
        loss_ref[...] = jnp.broadcast_to(loss, (1, 1))


def _pool_router(hidden_states, attention_mask, W1, b1, W2, b2):
    nsb = _S // _SB1
    return pl.pallas_call(
        _pool_router_body,
        grid=(nsb,),
        in_specs=[
            pl.BlockSpec((_B, _SB1, _D), lambda i: (0, i, 0)),
            pl.BlockSpec((_B, _SB1), lambda i: (0, i)),
            pl.BlockSpec((_D, _RH), lambda i: (0, 0)),
            pl.BlockSpec((1, _RH), lambda i: (0, 0)),
            pl.BlockSpec((_RH, _E), lambda i: (0, 0)),
            pl.BlockSpec((1, _E), lambda i: (0, 0)),
        ],
        out_specs=[
            pl.BlockSpec((_B, _E), lambda i: (0, 0)),
            pl.BlockSpec((_B, 1), lambda i: (0, 0)),
            pl.BlockSpec((1, 1), lambda i: (0, 0)),
        ],
        out_shape=[
            jax.ShapeDtypeStruct((_B, _E), jnp.float32),
            jax.ShapeDtypeStruct((_B, 1), jnp.int32),
            jax.ShapeDtypeStruct((1, 1), jnp.float32),
        ],
        scratch_shapes=[
            pltpu.VMEM((_B, _D), jnp.float32),
            pltpu.VMEM((_B, 128), jnp.float32),
        ],
        compiler_params=pltpu.CompilerParams(
            dimension_semantics=("arbitrary",)),
    )(hidden_states, attention_mask, W1, b1.reshape(1, _RH),
      W2, b2.reshape(1, _E))


_LANES = 16


def _sc_top1(rw_v):
    lane = lax.iota(jnp.int32, _LANES)
    ivec = jnp.zeros((_LANES,), jnp.int32)
    for b in range(_B):
        val = rw_v[b, pl.ds(0, _LANES)]
        idxv = lane
        for c in range(1, _E // _LANES):
            v = rw_v[b, pl.ds(c * _LANES, _LANES)]
            iv = lane + c * _LANES
            take = v > val
            val = jnp.where(take, v, val)
            idxv = jnp.where(take, iv, idxv)
        for sh in (8, 4, 2, 1):
            perm = lane ^ sh
            ov = val.at[perm].get(mode="promise_in_bounds")
            oi = idxv.at[perm].get(mode="promise_in_bounds")
            take = (ov > val) | ((ov == val) & (oi < idxv))
            val = jnp.where(take, ov, val)
            idxv = jnp.where(take, oi, idxv)
        ivec = jnp.where(lane == b, idxv, ivec)
    return ivec


def _sc_topk_kernel(rw_hbm, idx_hbm, rw_v, idx_v):
    w = lax.axis_index("s") * 2 + lax.axis_index("c")

    @pl.when(w == 0)
    def _select():
        pltpu.sync_copy(rw_hbm, rw_v)
        idx_v[...] = _sc_top1(rw_v)
        pltpu.sync_copy(idx_v, idx_hbm)


def _sc_topk(rw):
    mesh = plsc.VectorSubcoreMesh(core_axis_name="c", subcore_axis_name="s")
    fn = functools.partial(
        pl.kernel,
        mesh=mesh,
        out_type=jax.ShapeDtypeStruct((_LANES,), jnp.int32),
        scratch_types=[
            pltpu.VMEM((_B, _E), jnp.float32),
            pltpu.VMEM((_LANES,), jnp.int32),
        ],
    )(_sc_topk_kernel)
    return fn(rw)


def _apply_body(sidx_ref, h_ref, a_ref, b_ref, o_ref):
    del sidx_ref
    h = h_ref[0]
    low = lax.dot_general(h, a_ref[0], (((1,), (1,)), ((), ())),
                          preferred_element_type=jnp.float32)
    delta = lax.dot_general(low, b_ref[0], (((1,), (1,)), ((), ())),
                            preferred_element_type=jnp.float32)
    o_ref[0] = h + _SCALE * delta


def _apply(hidden_states, lora_A, lora_B, sidx):
    nsb = _S // _SB3
    grid_spec = pltpu.PrefetchScalarGridSpec(
        num_scalar_prefetch=1,
        grid=(_B, nsb),
        in_specs=[
            pl.BlockSpec((1, _SB3, _D), lambda b, s, sidx: (b, s, 0)),
            pl.BlockSpec((1, _R, _D), lambda b, s, sidx: (sidx[b], 0, 0)),
            pl.BlockSpec((1, _D, _R), lambda b, s, sidx: (sidx[b], 0, 0)),
        ],
        out_specs=pl.BlockSpec((1, _SB3, _D), lambda b, s, sidx: (b, s, 0)),
    )
    return pl.pallas_call(
        _apply_body,
        grid_spec=grid_spec,
        out_shape=jax.ShapeDtypeStruct((_B, _S, _D), jnp.float32),
        compiler_params=pltpu.CompilerParams(
            dimension_semantics=("arbitrary", "arbitrary")),
    )(sidx, hidden_states, lora_A, lora_B)


def kernel(input_ids, attention_mask, hidden_states, W1, b1, W2, b2,
           lora_A, lora_B):
    del input_ids
    rw, idx41, loss11 = _pool_router(hidden_states, attention_mask,
                                     W1, b1, W2, b2)
    out = _apply(hidden_states, lora_A, lora_B, idx41.reshape(_B))
    top_k_indices = idx41
    balance_loss = loss11[0, 0]
    return out, rw, top_k_indices, balance_loss
